# speedup vs baseline: 6.8994x; 6.8994x over previous
import jax
import jax.numpy as jnp
from jax.experimental import pallas as pl

_VOCAB = 32768
_ROWS_BLK = 32


def _onehot_body(ids_ref, vals_ref, out_ref):
    ids = ids_ref[...]
    vals = vals_ref[...]
    iota = jax.lax.broadcasted_iota(jnp.int32, (_ROWS_BLK, _VOCAB), 1)
    out_ref[...] = jnp.where(iota == ids, vals, jnp.float32(0.0))


def kernel(input_ids, attention_mask, boost):
    B, S = input_ids.shape
    N = B * S
    ids32 = input_ids.astype(jnp.int32)
    ids = jnp.clip(ids32, 0, _VOCAB - 1).reshape(N, 1)
    valid = (attention_mask == 1) & (ids32 >= 0) & (ids32 < _VOCAB)
    vals = jnp.where(valid.reshape(N, 1), boost.astype(jnp.float32),
                     jnp.float32(0.0))
    out = pl.pallas_call(
        _onehot_body,
        grid=(N // _ROWS_BLK,),
        in_specs=[
            pl.BlockSpec((_ROWS_BLK, 1), lambda i: (i, 0)),
            pl.BlockSpec((_ROWS_BLK, 1), lambda i: (i, 0)),
        ],
        out_specs=pl.BlockSpec((_ROWS_BLK, _VOCAB), lambda i: (i, 0)),
        out_shape=jax.ShapeDtypeStruct((N, _VOCAB), jnp.float32),
    )(ids, vals)
    return out.reshape(B, S, _VOCAB)
